# baseline (speedup 1.0000x reference)
import functools

import jax
import jax.numpy as jnp
from jax import lax
from jax.experimental import pallas as pl
from jax.experimental.pallas import tpu as pltpu
from jax.experimental.pallas import tpu_sc as plsc

_B, _L, _D = 4, 2048, 1024
_NC, _NS = 2, 16
_NW = _NC * _NS

_SC_ROWS = 512
_R1 = _B * _L - _SC_ROWS
_PBASE = _L - _SC_ROWS
_CH = 8
_RPW = _SC_ROWS // _NW
_NCHUNK = _RPW // _CH

_mesh = plsc.VectorSubcoreMesh(
    core_axis_name="c", subcore_axis_name="s", num_cores=_NC, num_subcores=_NS
)


@functools.partial(
    pl.kernel,
    out_type=jax.ShapeDtypeStruct((_SC_ROWS, _D), jnp.float32),
    mesh=_mesh,
    scratch_types=[
        pltpu.VMEM((_RPW, _D), jnp.float32),
        pltpu.VMEM((_CH, _D), jnp.float32),
        pltpu.VMEM((_CH, _D), jnp.float32),
        pltpu.SemaphoreType.DMA,
        pltpu.SemaphoreType.DMA,
        pltpu.SemaphoreType.DMA,
        pltpu.SemaphoreType.DMA,
        pltpu.SemaphoreType.DMA,
    ],
)
def _pos_add_sc(x_hbm, pos_hbm, out_hbm, pos_v, xa, xb,
                pos_sem, in0, in1, out0, out1):
    wid = lax.axis_index("s") * _NC + lax.axis_index("c")
    rbase = wid * _RPW
    bufs = (xa, xb)
    in_sems = (in0, in1)
    out_sems = (out0, out1)

    pos_cp = pltpu.make_async_copy(
        pos_hbm.at[pl.ds(_PBASE + rbase, _RPW), :], pos_v, pos_sem)
    pos_cp.start()

    loads = [
        pltpu.make_async_copy(
            x_hbm.at[pl.ds(_R1 + rbase + k * _CH, _CH), :], bufs[k % 2],
            in_sems[k % 2])
        for k in range(_NCHUNK)
    ]
    stores = [
        pltpu.make_async_copy(
            bufs[k % 2], out_hbm.at[pl.ds(rbase + k * _CH, _CH), :],
            out_sems[k % 2])
        for k in range(_NCHUNK)
    ]

    loads[0].start()
    for k in range(_NCHUNK):
        if k + 1 < _NCHUNK:
            if k >= 1:
                stores[k - 1].wait()
            loads[k + 1].start()
        loads[k].wait()
        if k == 0:
            pos_cp.wait()
        x_v = bufs[k % 2]
        prow = k * _CH

        @plsc.parallel_loop(0, _D, step=16, unroll=2)
        def _(i):
            for r in range(_CH):
                plsc.addupdate(x_v.at[r, pl.ds(i, 16)],
                               pos_v[prow + r, pl.ds(i, 16)])

        stores[k].start()
    stores[_NCHUNK - 2].wait()
    stores[_NCHUNK - 1].wait()


_TBLK = 512


def _tc_body(x_ref, p_hbm, o_ref, p_v, p_sem):
    i = pl.program_id(0)

    @pl.when(i == 0)
    def _():
        cp = pltpu.make_async_copy(p_hbm.at[pl.ds(0, _L), :], p_v, p_sem)
        cp.start()
        cp.wait()

    half = (i % (_L // _TBLK)) * _TBLK
    o_ref[...] = x_ref[...] + p_v[pl.ds(half, _TBLK), :]


_pos_add_tc = pl.pallas_call(
    _tc_body,
    grid=(_R1 // _TBLK,),
    in_specs=[
        pl.BlockSpec((_TBLK, _D), lambda i: (i, 0)),
        pl.BlockSpec(memory_space=pl.ANY),
    ],
    out_specs=pl.BlockSpec((_TBLK, _D), lambda i: (i, 0)),
    out_shape=jax.ShapeDtypeStruct((_B * _L, _D), jnp.float32),
    scratch_shapes=[
        pltpu.VMEM((_L, _D), jnp.float32),
        pltpu.SemaphoreType.DMA,
    ],
)


def kernel(x, pos_table):
    x2 = x.reshape(_B * _L, _D)
    out_sc = _pos_add_sc(x2, pos_table)
    out_tc = _pos_add_tc(x2, pos_table)
    out = lax.dynamic_update_slice(out_tc, out_sc, (_R1, 0))
    return out.reshape(x.shape)


# speedup vs baseline: 1.0040x; 1.0040x over previous
import functools

import jax
import jax.numpy as jnp
from jax import lax
from jax.experimental import pallas as pl
from jax.experimental.pallas import tpu as pltpu
from jax.experimental.pallas import tpu_sc as plsc

_B, _L, _D = 4, 2048, 1024
_NC, _NS = 2, 16
_NW = _NC * _NS

_SC_ROWS = 1024
_R1 = _B * _L - _SC_ROWS
_PBASE = _L - _SC_ROWS
_CH = 8
_RPW = _SC_ROWS // _NW
_NCHUNK = _RPW // _CH

_mesh = plsc.VectorSubcoreMesh(
    core_axis_name="c", subcore_axis_name="s", num_cores=_NC, num_subcores=_NS
)


@functools.partial(
    pl.kernel,
    out_type=jax.ShapeDtypeStruct((_SC_ROWS, _D), jnp.float32),
    mesh=_mesh,
    scratch_types=[
        pltpu.VMEM((_RPW, _D), jnp.float32),
        pltpu.VMEM((_CH, _D), jnp.float32),
        pltpu.VMEM((_CH, _D), jnp.float32),
        pltpu.SemaphoreType.DMA,
        pltpu.SemaphoreType.DMA,
        pltpu.SemaphoreType.DMA,
        pltpu.SemaphoreType.DMA,
        pltpu.SemaphoreType.DMA,
    ],
)
def _pos_add_sc(x_hbm, pos_hbm, out_hbm, pos_v, xa, xb,
                pos_sem, in0, in1, out0, out1):
    wid = lax.axis_index("s") * _NC + lax.axis_index("c")
    rbase = wid * _RPW
    bufs = (xa, xb)
    in_sems = (in0, in1)
    out_sems = (out0, out1)

    pos_cp = pltpu.make_async_copy(
        pos_hbm.at[pl.ds(_PBASE + rbase, _RPW), :], pos_v, pos_sem)
    pos_cp.start()

    loads = [
        pltpu.make_async_copy(
            x_hbm.at[pl.ds(_R1 + rbase + k * _CH, _CH), :], bufs[k % 2],
            in_sems[k % 2])
        for k in range(_NCHUNK)
    ]
    stores = [
        pltpu.make_async_copy(
            bufs[k % 2], out_hbm.at[pl.ds(rbase + k * _CH, _CH), :],
            out_sems[k % 2])
        for k in range(_NCHUNK)
    ]

    loads[0].start()
    for k in range(_NCHUNK):
        if k + 1 < _NCHUNK:
            if k >= 1:
                stores[k - 1].wait()
            loads[k + 1].start()
        loads[k].wait()
        if k == 0:
            pos_cp.wait()
        x_v = bufs[k % 2]
        prow = k * _CH

        @plsc.parallel_loop(0, _D, step=16, unroll=2)
        def _(i):
            for r in range(_CH):
                plsc.addupdate(x_v.at[r, pl.ds(i, 16)],
                               pos_v[prow + r, pl.ds(i, 16)])

        stores[k].start()
    stores[_NCHUNK - 2].wait()
    stores[_NCHUNK - 1].wait()


_TBLK = 1024


def _tc_body(x_ref, p_hbm, o_ref, p_v, p_sem0, p_sem1):
    i = pl.program_id(0)

    def _half_cp(h, sem):
        return pltpu.make_async_copy(
            p_hbm.at[pl.ds(h * _TBLK, _TBLK), :],
            p_v.at[pl.ds(h * _TBLK, _TBLK), :], sem)

    @pl.when(i == 0)
    def _():
        _half_cp(0, p_sem0).start()
        _half_cp(1, p_sem1).start()
        _half_cp(0, p_sem0).wait()

    @pl.when(i == 1)
    def _():
        _half_cp(1, p_sem1).wait()

    half = (i % (_L // _TBLK)) * _TBLK
    o_ref[...] = x_ref[...] + p_v[pl.ds(half, _TBLK), :]


_pos_add_tc = pl.pallas_call(
    _tc_body,
    grid=(_R1 // _TBLK,),
    in_specs=[
        pl.BlockSpec((_TBLK, _D), lambda i: (i, 0)),
        pl.BlockSpec(memory_space=pl.ANY),
    ],
    out_specs=pl.BlockSpec((_TBLK, _D), lambda i: (i, 0)),
    out_shape=jax.ShapeDtypeStruct((_B * _L, _D), jnp.float32),
    scratch_shapes=[
        pltpu.VMEM((_L, _D), jnp.float32),
        pltpu.SemaphoreType.DMA,
        pltpu.SemaphoreType.DMA,
    ],
)


def kernel(x, pos_table):
    x2 = x.reshape(_B * _L, _D)
    out_sc = _pos_add_sc(x2, pos_table)
    out_tc = _pos_add_tc(x2, pos_table)
    out = lax.dynamic_update_slice(out_tc, out_sc, (_R1, 0))
    return out.reshape(x.shape)
